# Initial kernel scaffold; baseline (speedup 1.0000x reference)
import jax
import jax.numpy as jnp
from jax.experimental import pallas as pl


def kernel(table, x):
    raise NotImplementedError("write your pallas kernel here")


# speedup vs baseline: 3.2367x; 3.2367x over previous
import jax
import jax.numpy as jnp
from jax import lax
from jax.experimental import pallas as pl
from jax.experimental.pallas import tpu as pltpu
from jax.experimental.pallas import tpu_sc as plsc

EMB = 100
PAD = 128
NUM_CORES = 2
NUM_SUBCORES = 16
NW = NUM_CORES * NUM_SUBCORES
CHUNK = 128
_COPY_OFFS = (0, 16, 32, 48, 64, 80, 84)


def _gather_body(table_hbm, idx_hbm, out_hbm, idx_v, pad_v, rows_v, sem_g):
    n_chunks = idx_hbm.shape[1]
    wid = lax.axis_index("s") * NUM_CORES + lax.axis_index("c")
    pltpu.sync_copy(idx_hbm.at[wid], idx_v)

    def chunk_body(j, carry):
        pltpu.async_copy(table_hbm.at[idx_v.at[j]], pad_v, sem_g).wait()

        def row_body(r, c):
            for off in _COPY_OFFS:
                rows_v[r, pl.ds(off, 16)] = pad_v[r, pl.ds(off, 16)]
            return c

        lax.fori_loop(0, CHUNK, row_body, 0)
        pltpu.sync_copy(rows_v, out_hbm.at[wid, j])
        return carry

    lax.fori_loop(0, n_chunks, chunk_body, 0)


@jax.jit
def _run(table_padded, xr):
    n_chunks = xr.shape[1]
    mesh = plsc.VectorSubcoreMesh(core_axis_name="c", subcore_axis_name="s")
    f = pl.kernel(
        _gather_body,
        mesh=mesh,
        out_type=jax.ShapeDtypeStruct((NW, n_chunks, CHUNK, EMB), jnp.float32),
        scratch_types=[
            pltpu.VMEM((n_chunks, CHUNK), jnp.int32),
            pltpu.VMEM((CHUNK, PAD), jnp.float32),
            pltpu.VMEM((CHUNK, EMB), jnp.float32),
            pltpu.SemaphoreType.DMA,
        ],
    )
    return f(table_padded, xr)


def kernel(table, x):
    b, h = x.shape
    n = b * h
    assert n % (NW * CHUNK) == 0
    n_chunks = n // (NW * CHUNK)
    tp = jnp.pad(table, ((0, 0), (0, PAD - EMB)))
    xr = x.astype(jnp.int32).reshape(NW, n_chunks, CHUNK)
    out = _run(tp, xr)
    return out.reshape(b, h, EMB)


# speedup vs baseline: 4.5090x; 1.3931x over previous
import jax
import jax.numpy as jnp
from jax import lax
from jax.experimental import pallas as pl
from jax.experimental.pallas import tpu as pltpu
from jax.experimental.pallas import tpu_sc as plsc

EMB = 100
PAD = 128
NUM_CORES = 2
NUM_SUBCORES = 16
NW = NUM_CORES * NUM_SUBCORES
ROWS_PER_WORKER = 128
CHUNK_A = 128
_COPY_OFFS = (0, 16, 32, 48, 64, 80, 84)


def _pad_body(t_ref, o_ref):
    o_ref[:, :EMB] = t_ref[...]


def _pad_table(table):
    v = table.shape[0]
    blk = 2000
    return pl.pallas_call(
        _pad_body,
        grid=(v // blk,),
        in_specs=[pl.BlockSpec((blk, EMB), lambda i: (i, 0))],
        out_specs=pl.BlockSpec((blk, PAD), lambda i: (i, 0)),
        out_shape=jax.ShapeDtypeStruct((v, PAD), jnp.float32),
    )(table)


def _gather_body(table_hbm, x_hbm, out_hbm, idx_v, pad_v, rows_v, sems):
    hist = x_hbm.shape[1]
    chunk_b = hist - CHUNK_A
    wid = lax.axis_index("s") * NUM_CORES + lax.axis_index("c")
    sem_ga, sem_gb, sem_wa, sem_wb = sems
    pltpu.sync_copy(x_hbm.at[pl.ds(wid * ROWS_PER_WORKER, ROWS_PER_WORKER)],
                    idx_v)

    def compact(src, dst, nrows):
        def row_body(r, c):
            for off in _COPY_OFFS:
                dst[r, pl.ds(off, 16)] = src[r, pl.ds(off, 16)]
            return c
        lax.fori_loop(0, nrows, row_body, 0)

    def g_a(r):
        return pltpu.make_async_copy(
            table_hbm.at[idx_v.at[r, pl.ds(0, CHUNK_A)]], pad_v.at[0], sem_ga)

    def g_b(r):
        return pltpu.make_async_copy(
            table_hbm.at[idx_v.at[r, pl.ds(CHUNK_A, chunk_b)]],
            pad_v.at[1, pl.ds(0, chunk_b)], sem_gb)

    def w_a(r):
        return pltpu.make_async_copy(
            rows_v.at[0], out_hbm.at[wid, r, pl.ds(0, CHUNK_A)], sem_wa)

    def w_b(r):
        return pltpu.make_async_copy(
            rows_v.at[1, pl.ds(0, chunk_b)],
            out_hbm.at[wid, r, pl.ds(CHUNK_A, chunk_b)], sem_wb)

    def body(r, carry):
        g_a(r).start()
        g_b(r).start()
        g_a(r).wait()
        pl.when(r > 0)(lambda: w_a(r - 1).wait())
        compact(pad_v.at[0], rows_v.at[0], CHUNK_A)
        w_a(r).start()
        g_b(r).wait()
        pl.when(r > 0)(lambda: w_b(r - 1).wait())
        compact(pad_v.at[1], rows_v.at[1], chunk_b)
        w_b(r).start()
        return carry

    lax.fori_loop(0, ROWS_PER_WORKER, body, 0)
    w_a(ROWS_PER_WORKER - 1).wait()
    w_b(ROWS_PER_WORKER - 1).wait()


@jax.jit
def _run(table, x):
    hist = x.shape[1]
    mesh = plsc.VectorSubcoreMesh(core_axis_name="c", subcore_axis_name="s")
    table_padded = _pad_table(table)
    f = pl.kernel(
        _gather_body,
        mesh=mesh,
        out_type=jax.ShapeDtypeStruct(
            (NW, ROWS_PER_WORKER, hist, EMB), jnp.float32),
        scratch_types=[
            pltpu.VMEM((ROWS_PER_WORKER, hist), jnp.int32),
            pltpu.VMEM((2, CHUNK_A, PAD), jnp.float32),
            pltpu.VMEM((2, CHUNK_A, EMB), jnp.float32),
            [pltpu.SemaphoreType.DMA] * 4,
        ],
    )
    return f(table_padded, x)


def kernel(table, x):
    b, h = x.shape
    assert b == NW * ROWS_PER_WORKER and CHUNK_A < h <= 2 * CHUNK_A
    out = _run(table, x.astype(jnp.int32))
    return out.reshape(b, h, EMB)
